# Initial kernel scaffold; baseline (speedup 1.0000x reference)
import jax
import jax.numpy as jnp
from jax.experimental import pallas as pl


def kernel(feat, edge_index, W_theta, b_theta, W_phi, b_phi):
    raise NotImplementedError("write your pallas kernel here")


# speedup vs baseline: 2.3173x; 2.3173x over previous
import functools

import jax
import jax.numpy as jnp
from jax import lax
from jax.experimental import pallas as pl
from jax.experimental.pallas import tpu as pltpu
from jax.experimental.pallas import tpu_sc as plsc

N_NODES = 10000
D = 128
E_TOTAL = 320000

NC = 2
NS = 16
NW = NC * NS
NPW = 320
N_PAD = NW * NPW
C = 6400
NCHUNK = E_TOTAL // C
NEG_INF = float("-inf")


def _splat_lane(vec, i):
    dnums = lax.GatherDimensionNumbers(
        offset_dims=(), collapsed_slice_dims=(0,), start_index_map=(0,))
    idx = jnp.full((16,), i, jnp.int32)
    return lax.gather(vec, idx[:, None], dnums, slice_sizes=(1,),
                      mode=lax.GatherScatterMode.PROMISE_IN_BOUNDS)


def _tc_linear(feat_pad, wt_t, wc_t, bias8):
    blk = 1024

    def body(x_ref, wt_ref, wc_ref, b_ref, a_ref, bm_ref):
        x = x_ref[...]
        a_ref[...] = (
            jnp.dot(x, wt_ref[...], preferred_element_type=jnp.float32)
            + b_ref[0:1, :]
        )
        bm_ref[...] = jnp.dot(x, wc_ref[...], preferred_element_type=jnp.float32)

    return pl.pallas_call(
        body,
        grid=(N_PAD // blk,),
        in_specs=[
            pl.BlockSpec((blk, D), lambda i: (i, 0)),
            pl.BlockSpec((D, D), lambda i: (0, 0)),
            pl.BlockSpec((D, D), lambda i: (0, 0)),
            pl.BlockSpec((8, D), lambda i: (0, 0)),
        ],
        out_specs=[
            pl.BlockSpec((blk, D), lambda i: (i, 0)),
            pl.BlockSpec((blk, D), lambda i: (i, 0)),
        ],
        out_shape=[
            jax.ShapeDtypeStruct((N_PAD, D), jnp.float32),
            jax.ShapeDtypeStruct((N_PAD, D), jnp.float32),
        ],
    )(feat_pad, wt_t, wc_t, bias8)


def _sc_segmax(a_flat, b_mat, src, dst):
    mesh = plsc.VectorSubcoreMesh(core_axis_name="c", subcore_axis_name="s")

    @functools.partial(
        pl.kernel,
        out_type=jax.ShapeDtypeStruct((N_PAD * D,), jnp.float32),
        mesh=mesh,
        scratch_types=[
            pltpu.VMEM(((NPW + 1) * D,), jnp.float32),
            pltpu.VMEM((NPW * D,), jnp.float32),
            pltpu.VMEM((C,), jnp.int32),
            pltpu.VMEM((C,), jnp.int32),
            pltpu.VMEM((C + 16,), jnp.int32),
            pltpu.VMEM((C + 16,), jnp.int32),
            pltpu.VMEM((16, D), jnp.float32),
            pltpu.SemaphoreType.DMA,
        ],
        compiler_params=pltpu.CompilerParams(needs_layout_passes=False),
    )
    def kern(a_hbm, b_hbm, src_hbm, dst_hbm, out_hbm,
             acc, stage, srcv, dstv, psrc, pldst, rows, sem):
        wid = lax.axis_index("s") * NC + lax.axis_index("c")
        base = wid * NPW
        iota = lax.iota(jnp.int32, 16)

        def init_body(i, carry):
            acc[pl.ds(i * 16, 16)] = jnp.full((16,), NEG_INF, jnp.float32)
            return carry
        lax.fori_loop(0, NPW * D // 16, init_body, 0)

        def chunk_body(ci, carry):
            pltpu.sync_copy(src_hbm.at[pl.ds(ci * C, C)], srcv)
            pltpu.sync_copy(dst_hbm.at[pl.ds(ci * C, C)], dstv)

            def scan_body(k, cnt):
                d = dstv[pl.ds(k * 16, 16)]
                s = srcv[pl.ds(k * 16, 16)]
                loc = d - base
                m = (loc >= 0) & (loc < NPW)
                incl = plsc.cumsum(m.astype(jnp.int32))
                pos = cnt + incl - 1
                plsc.store_scatter(psrc, [pos], s, mask=m)
                plsc.store_scatter(pldst, [pos], loc, mask=m)
                return cnt + jnp.max(incl)

            cnt = lax.fori_loop(0, C // 16, scan_body, jnp.int32(0))

            padpos = cnt + iota
            plsc.store_scatter(psrc, [padpos], base + iota)
            plsc.store_scatter(pldst, [padpos], jnp.full((16,), NPW, jnp.int32))
            nb = (cnt + 15) // 16

            def batch_body(bi, carry2):
                idxv = psrc[pl.ds(bi * 16, 16)]
                ldv = pldst[pl.ds(bi * 16, 16)]
                pltpu.async_copy(b_hbm.at[idxv], rows, sem).wait()
                for i in range(16):
                    ld = _splat_lane(ldv, i)
                    for j in range(D // 16):
                        addr = ld * D + (j * 16) + iota
                        aval = plsc.load_gather(acc, [addr])
                        rval = plsc.load_gather(
                            rows, [jnp.full((16,), i, jnp.int32), j * 16 + iota])
                        plsc.store_scatter(acc, [addr], jnp.maximum(aval, rval))
                return carry2

            lax.fori_loop(0, nb, batch_body, 0)
            return carry

        lax.fori_loop(0, NCHUNK, chunk_body, 0)

        pltpu.sync_copy(a_hbm.at[pl.ds(base * D, NPW * D)], stage)

        def ep_body(i, carry):
            mx = acc[pl.ds(i * 16, 16)]
            av = stage[pl.ds(i * 16, 16)]
            stage[pl.ds(i * 16, 16)] = jnp.where(
                mx > NEG_INF, mx + av, jnp.zeros((16,), jnp.float32))
            return carry
        lax.fori_loop(0, NPW * D // 16, ep_body, 0)

        pltpu.sync_copy(stage, out_hbm.at[pl.ds(base * D, NPW * D)])

    return kern(a_flat, b_mat, src, dst)


def kernel(feat, edge_index, W_theta, b_theta, W_phi, b_phi):
    src = edge_index[0].astype(jnp.int32)
    dst = edge_index[1].astype(jnp.int32)
    feat_pad = jnp.pad(feat, ((0, N_PAD - N_NODES), (0, 0)))
    wt_t = W_theta.T
    wc_t = (W_phi - W_theta).T
    bias8 = jnp.broadcast_to((b_theta + b_phi)[None, :], (8, D))
    a_mat, b_mat = _tc_linear(feat_pad, wt_t, wc_t, bias8)
    out_flat = _sc_segmax(a_mat.reshape(-1), b_mat, src, dst)
    return out_flat.reshape(N_PAD, D)[:N_NODES]


# speedup vs baseline: 4.1556x; 1.7933x over previous
import functools

import jax
import jax.numpy as jnp
from jax import lax
from jax.experimental import pallas as pl
from jax.experimental.pallas import tpu as pltpu
from jax.experimental.pallas import tpu_sc as plsc

N_NODES = 10000
D = 128
E_TOTAL = 320000
E_HALF = E_TOTAL // 2

NC = 2
NS = 16
NPW = 640
N_PAD = NS * NPW
C = 6400
NCHUNK = E_HALF // C
NEG_INF = float("-inf")


def _tc_b(feat_pad, wc_t):
    blk = 1024

    def body(x_ref, wc_ref, bm_ref):
        bm_ref[...] = jnp.dot(x_ref[...], wc_ref[...],
                              preferred_element_type=jnp.float32)

    return pl.pallas_call(
        body,
        grid=(N_PAD // blk,),
        in_specs=[
            pl.BlockSpec((blk, D), lambda i: (i, 0)),
            pl.BlockSpec((D, D), lambda i: (0, 0)),
        ],
        out_specs=pl.BlockSpec((blk, D), lambda i: (i, 0)),
        out_shape=jax.ShapeDtypeStruct((N_PAD, D), jnp.float32),
    )(feat_pad, wc_t)


def _tc_epilogue(partials, feat_pad, wt_t, bias8):
    blk = 1024

    def body(p_ref, x_ref, wt_ref, b_ref, o_ref):
        m = jnp.maximum(p_ref[0], p_ref[1])
        a = (jnp.dot(x_ref[...], wt_ref[...],
                     preferred_element_type=jnp.float32) + b_ref[0:1, :])
        o_ref[...] = jnp.where(m > NEG_INF, m + a, 0.0)

    return pl.pallas_call(
        body,
        grid=(N_PAD // blk,),
        in_specs=[
            pl.BlockSpec((2, blk, D), lambda i: (0, i, 0)),
            pl.BlockSpec((blk, D), lambda i: (i, 0)),
            pl.BlockSpec((D, D), lambda i: (0, 0)),
            pl.BlockSpec((8, D), lambda i: (0, 0)),
        ],
        out_specs=pl.BlockSpec((blk, D), lambda i: (i, 0)),
        out_shape=jax.ShapeDtypeStruct((N_PAD, D), jnp.float32),
    )(partials, feat_pad, wt_t, bias8)


def _sc_segmax(b_mat, src, dst):
    mesh = plsc.VectorSubcoreMesh(core_axis_name="c", subcore_axis_name="s")

    @functools.partial(
        pl.kernel,
        out_type=jax.ShapeDtypeStruct((NC * N_PAD * D,), jnp.float32),
        mesh=mesh,
        scratch_types=[
            pltpu.VMEM(((NPW + 1) * D,), jnp.float32),
            pltpu.VMEM((C,), jnp.int32),
            pltpu.VMEM((C,), jnp.int32),
            pltpu.VMEM((C + 32,), jnp.int32),
            pltpu.VMEM((C + 32,), jnp.int32),
            pltpu.VMEM((16, D), jnp.float32),
            pltpu.VMEM((16, D), jnp.float32),
            pltpu.SemaphoreType.DMA,
            pltpu.SemaphoreType.DMA,
        ],
        compiler_params=pltpu.CompilerParams(needs_layout_passes=False),
    )
    def kern(b_hbm, src_hbm, dst_hbm, out_hbm,
             acc, srcv, dstv, psrc, pldst, rows0, rows1, sem0, sem1):
        core = lax.axis_index("c")
        sub = lax.axis_index("s")
        base = sub * NPW
        ebase = core * E_HALF
        iota = lax.iota(jnp.int32, 16)

        def init_body(i, carry):
            acc[pl.ds(i * 16, 16)] = jnp.full((16,), NEG_INF, jnp.float32)
            return carry
        lax.fori_loop(0, NPW * D // 16, init_body, 0)

        def fire(b, buf, sem):
            idxv = psrc[pl.ds(b * 16, 16)]
            pltpu.async_copy(b_hbm.at[idxv], buf, sem)

        def wait(b, buf, sem):
            idxv = psrc[pl.ds(b * 16, 16)]
            pltpu.make_async_copy(b_hbm.at[idxv], buf, sem).wait()

        def apply16(b, buf):
            ldv = pldst[pl.ds(b * 16, 16)]
            for i in range(16):
                off = ldv[i] * D
                for j in range(D // 16):
                    a = acc[pl.ds(off + j * 16, 16)]
                    r = buf[i, pl.ds(j * 16, 16)]
                    acc[pl.ds(off + j * 16, 16)] = jnp.maximum(a, r)

        def chunk_body(ci, carry):
            pltpu.sync_copy(src_hbm.at[pl.ds(ebase + ci * C, C)], srcv)
            pltpu.sync_copy(dst_hbm.at[pl.ds(ebase + ci * C, C)], dstv)

            def scan_body(k, cnt):
                d = dstv[pl.ds(k * 16, 16)]
                s = srcv[pl.ds(k * 16, 16)]
                loc = d - base
                m = (loc >= 0) & (loc < NPW)
                incl = plsc.cumsum(m.astype(jnp.int32))
                pos = cnt + incl - 1
                plsc.store_scatter(psrc, [pos], s, mask=m)
                plsc.store_scatter(pldst, [pos], loc, mask=m)
                return cnt + incl[15]

            cnt = lax.fori_loop(0, C // 16, scan_body, jnp.int32(0))

            trash_ld = jnp.full((16,), NPW, jnp.int32)
            plsc.store_scatter(psrc, [cnt + iota], base + iota)
            plsc.store_scatter(pldst, [cnt + iota], trash_ld)
            plsc.store_scatter(psrc, [cnt + 16 + iota], base + 16 + iota)
            plsc.store_scatter(pldst, [cnt + 16 + iota], trash_ld)
            npairs = (cnt + 31) // 32

            @pl.when(npairs > 0)
            def _():
                fire(0, rows0, sem0)

            def pair_body(g, carry2):
                wait(2 * g, rows0, sem0)
                fire(2 * g + 1, rows1, sem1)
                apply16(2 * g, rows0)
                wait(2 * g + 1, rows1, sem1)

                @pl.when(g + 1 < npairs)
                def _():
                    fire(2 * g + 2, rows0, sem0)

                apply16(2 * g + 1, rows1)
                return carry2

            lax.fori_loop(0, npairs, pair_body, 0)
            return carry

        lax.fori_loop(0, NCHUNK, chunk_body, 0)

        pltpu.sync_copy(
            acc.at[pl.ds(0, NPW * D)],
            out_hbm.at[pl.ds(core * (N_PAD * D) + base * D, NPW * D)])

    return kern(b_mat, src, dst)


def kernel(feat, edge_index, W_theta, b_theta, W_phi, b_phi):
    src = edge_index[0].astype(jnp.int32)
    dst = edge_index[1].astype(jnp.int32)
    feat_pad = jnp.pad(feat, ((0, N_PAD - N_NODES), (0, 0)))
    wt_t = W_theta.T
    wc_t = (W_phi - W_theta).T
    bias8 = jnp.broadcast_to((b_theta + b_phi)[None, :], (8, D))
    b_mat = _tc_b(feat_pad, wc_t)
    partials = _sc_segmax(b_mat, src, dst).reshape(NC, N_PAD, D)
    out = _tc_epilogue(partials, feat_pad, wt_t, bias8)
    return out[:N_NODES]
